# baseline (speedup 1.0000x reference)
import functools

import jax
import jax.numpy as jnp
from jax import lax
from jax.experimental import pallas as pl
from jax.experimental.pallas import tpu as pltpu
from jax.experimental.pallas import tpu_sc as plsc

_NC = 2
_NS = 16
_NW = _NC * _NS

_VOC = 1000000
_D = 32
_B = 16384
_F = 26

_TOT = _B * _F
_IDX_MINOR = 128
_IDX_ROWS = _TOT // _IDX_MINOR
_ROWS_PER_W = _IDX_ROWS // _NW
_CHUNK_IR = 8
_CHUNK = _CHUNK_IR * _IDX_MINOR
_STEPS = _ROWS_PER_W // _CHUNK_IR
_T1_C = 512
_T1_GRID = 489
_ST = _T1_C * _T1_GRID
_T1_MAXB = _VOC // _T1_C


def _tc_emb_pack(embT):
    def body(i0, i1, i2, i3, out_ref):
        parts = [r[...].T for r in (i0, i1, i2, i3)]
        out_ref[...] = jnp.concatenate(parts, axis=1)

    def imap(k):
        return lambda c: (0, jnp.minimum(k * _T1_GRID + c, _T1_MAXB))

    return pl.pallas_call(
        body,
        grid=(_T1_GRID,),
        in_specs=[pl.BlockSpec((_D, _T1_C), imap(k)) for k in range(4)],
        out_specs=pl.BlockSpec((_T1_C, 128), lambda c: (c, 0)),
        out_shape=jax.ShapeDtypeStruct((_ST, 128), jnp.float32),
    )(embT, embT, embT, embT)


def _sc_gather(emb, idx2d):
    mesh = plsc.VectorSubcoreMesh(
        core_axis_name="c", subcore_axis_name="s",
        num_cores=_NC, num_subcores=_NS)

    @functools.partial(
        pl.kernel,
        out_type=jax.ShapeDtypeStruct((_TOT, _D), jnp.float32),
        mesh=mesh,
        scratch_types=[
            pltpu.VMEM((2, _CHUNK_IR, _IDX_MINOR), jnp.int32),
            pltpu.VMEM((2, _CHUNK, _D), jnp.float32),
            pltpu.SemaphoreType.DMA,
            pltpu.SemaphoreType.DMA,
            pltpu.SemaphoreType.DMA,
            pltpu.SemaphoreType.DMA,
        ],
        compiler_params=pltpu.CompilerParams(use_tc_tiling_on_sc=False),
    )
    def k(emb_hbm, idx_hbm, out_hbm, idx_v, rows_v, g0, g1, w0, w1):
        wid = lax.axis_index("s") * _NC + lax.axis_index("c")
        base_ir = wid * _ROWS_PER_W
        gsem = (g0, g1)
        wsem = (w0, w1)

        gathers = [None, None]
        writes = [None, None]
        for t in range(_STEPS):
            buf = t % 2
            if writes[buf] is not None:
                writes[buf].wait()
            ir0 = base_ir + t * _CHUNK_IR
            pltpu.sync_copy(idx_hbm.at[pl.ds(ir0, _CHUNK_IR)],
                            idx_v.at[buf])
            gathers[buf] = [
                pltpu.async_copy(
                    emb_hbm.at[idx_v.at[buf, j]],
                    rows_v.at[buf, pl.ds(j * _IDX_MINOR, _IDX_MINOR)],
                    gsem[buf])
                for j in range(_CHUNK_IR)
            ]
            if t >= 1:
                pb = 1 - buf
                for c in gathers[pb]:
                    c.wait()
                pr0 = (base_ir + (t - 1) * _CHUNK_IR) * _IDX_MINOR
                writes[pb] = pltpu.async_copy(
                    rows_v.at[pb], out_hbm.at[pl.ds(pr0, _CHUNK)], wsem[pb])
        last = (_STEPS - 1) % 2
        for c in gathers[last]:
            c.wait()
        lr0 = (base_ir + (_STEPS - 1) * _CHUNK_IR) * _IDX_MINOR
        writes[last] = pltpu.async_copy(
            rows_v.at[last], out_hbm.at[pl.ds(lr0, _CHUNK)], wsem[last])
        writes[0].wait()
        writes[1].wait()

    return k(emb, idx2d)


_TC_BLK_B = 16384
_TC_M = _TC_BLK_B // 4
_TC_NBLK = _B // _TC_BLK_B


def _tc_sigmoid_transpose(flat):
    def body(in_ref, out_ref):
        v2 = in_ref[...].reshape(_TC_M, 128)
        parts = [v2[:, 32 * k:32 * (k + 1)].T for k in range(4)]
        out_ref[0] = jax.nn.sigmoid(jnp.concatenate(parts, axis=1))

    return pl.pallas_call(
        body,
        grid=(_F, _TC_NBLK),
        in_specs=[pl.BlockSpec((_TC_BLK_B * _D,),
                               lambda f, c: (f * _TC_NBLK + c,))],
        out_specs=pl.BlockSpec((1, _D, _TC_BLK_B), lambda f, c: (f, 0, c)),
        out_shape=jax.ShapeDtypeStruct((_F, _D, _B), jnp.float32),
    )(flat)


@jax.jit
def kernel(x, emb):
    xt = jnp.transpose(x).astype(jnp.int32)
    k = xt // _ST
    xt = (xt - k * _ST) * 4 + k
    idx2d = (xt.reshape(_F, _TC_NBLK, 4, _TC_M)
             .transpose(0, 1, 3, 2)
             .reshape(_IDX_ROWS, _IDX_MINOR))
    emb_pack = _tc_emb_pack(jnp.transpose(emb))
    rows = _sc_gather(emb_pack.reshape(_ST * 4, _D), idx2d)
    out3 = _tc_sigmoid_transpose(rows.reshape(-1))
    return out3.transpose(2, 0, 1)


# speedup vs baseline: 1.3169x; 1.3169x over previous
import functools

import jax
import jax.numpy as jnp
from jax import lax
from jax.experimental import pallas as pl
from jax.experimental.pallas import tpu as pltpu
from jax.experimental.pallas import tpu_sc as plsc

_NC = 2
_NS = 16
_NW = _NC * _NS

_VOC = 1000000
_D = 32
_B = 16384
_F = 26

_TOT = _B * _F
_IDX_MINOR = 128
_IDX_ROWS = _TOT // _IDX_MINOR
_ROWS_PER_W = _IDX_ROWS // _NW
_CHUNK_IR = 8
_CHUNK = _CHUNK_IR * _IDX_MINOR
_STEPS = _ROWS_PER_W // _CHUNK_IR
_T1_C = 2048
_T1_GRID = 123
_ST = _T1_C * _T1_GRID
_T1_MAXB = _VOC // _T1_C


def _tc_emb_pack(embT):
    def body(i0, i1, i2, i3, out_ref):
        parts = [r[...].T for r in (i0, i1, i2, i3)]
        out_ref[...] = jnp.concatenate(parts, axis=1)

    def imap(k):
        return lambda c: (0, jnp.minimum(k * _T1_GRID + c, _T1_MAXB))

    return pl.pallas_call(
        body,
        grid=(_T1_GRID,),
        in_specs=[pl.BlockSpec((_D, _T1_C), imap(k)) for k in range(4)],
        out_specs=pl.BlockSpec((_T1_C, 128), lambda c: (c, 0)),
        out_shape=jax.ShapeDtypeStruct((_ST, 128), jnp.float32),
    )(embT, embT, embT, embT)


def _sc_gather(emb, idx2d):
    mesh = plsc.VectorSubcoreMesh(
        core_axis_name="c", subcore_axis_name="s",
        num_cores=_NC, num_subcores=_NS)

    @functools.partial(
        pl.kernel,
        out_type=jax.ShapeDtypeStruct((_TOT, _D), jnp.float32),
        mesh=mesh,
        scratch_types=[
            pltpu.VMEM((2, _CHUNK_IR, _IDX_MINOR), jnp.int32),
            pltpu.VMEM((2, _CHUNK, _D), jnp.float32),
            pltpu.SemaphoreType.DMA,
            pltpu.SemaphoreType.DMA,
            pltpu.SemaphoreType.DMA,
            pltpu.SemaphoreType.DMA,
        ],
        compiler_params=pltpu.CompilerParams(use_tc_tiling_on_sc=False),
    )
    def k(emb_hbm, idx_hbm, out_hbm, idx_v, rows_v, g0, g1, w0, w1):
        wid = lax.axis_index("s") * _NC + lax.axis_index("c")
        base_ir = wid * _ROWS_PER_W
        gsem = (g0, g1)
        wsem = (w0, w1)

        gathers = [None, None]
        writes = [None, None]
        for t in range(_STEPS):
            buf = t % 2
            if writes[buf] is not None:
                writes[buf].wait()
            ir0 = base_ir + t * _CHUNK_IR
            pltpu.sync_copy(idx_hbm.at[pl.ds(ir0, _CHUNK_IR)],
                            idx_v.at[buf])
            gathers[buf] = [
                pltpu.async_copy(
                    emb_hbm.at[idx_v.at[buf, j]],
                    rows_v.at[buf, pl.ds(j * _IDX_MINOR, _IDX_MINOR)],
                    gsem[buf])
                for j in range(_CHUNK_IR)
            ]
            if t >= 1:
                pb = 1 - buf
                for c in gathers[pb]:
                    c.wait()
                pr0 = (base_ir + (t - 1) * _CHUNK_IR) * _IDX_MINOR
                writes[pb] = pltpu.async_copy(
                    rows_v.at[pb], out_hbm.at[pl.ds(pr0, _CHUNK)], wsem[pb])
        last = (_STEPS - 1) % 2
        for c in gathers[last]:
            c.wait()
        lr0 = (base_ir + (_STEPS - 1) * _CHUNK_IR) * _IDX_MINOR
        writes[last] = pltpu.async_copy(
            rows_v.at[last], out_hbm.at[pl.ds(lr0, _CHUNK)], wsem[last])
        writes[0].wait()
        writes[1].wait()

    return k(emb, idx2d)


_TC_BLK_B = 16384
_TC_M = _TC_BLK_B // 4
_TC_NBLK = _B // _TC_BLK_B


def _tc_sigmoid_transpose(flat):
    def body(in_ref, out_ref):
        v2 = in_ref[...].reshape(_TC_M, 128)
        parts = [v2[:, 32 * k:32 * (k + 1)].T for k in range(4)]
        out_ref[0] = jax.nn.sigmoid(jnp.concatenate(parts, axis=1))

    return pl.pallas_call(
        body,
        grid=(_F, _TC_NBLK),
        in_specs=[pl.BlockSpec((_TC_BLK_B * _D,),
                               lambda f, c: (f * _TC_NBLK + c,))],
        out_specs=pl.BlockSpec((1, _D, _TC_BLK_B), lambda f, c: (f, 0, c)),
        out_shape=jax.ShapeDtypeStruct((_F, _D, _B), jnp.float32),
    )(flat)


@jax.jit
def kernel(x, emb):
    xt = jnp.transpose(x).astype(jnp.int32)
    k = xt // _ST
    xt = (xt - k * _ST) * 4 + k
    idx2d = (xt.reshape(_F, _TC_NBLK, 4, _TC_M)
             .transpose(0, 1, 3, 2)
             .reshape(_IDX_ROWS, _IDX_MINOR))
    emb_pack = _tc_emb_pack(jnp.transpose(emb))
    rows = _sc_gather(emb_pack.reshape(_ST * 4, _D), idx2d)
    out3 = _tc_sigmoid_transpose(rows.reshape(-1))
    return out3.transpose(2, 0, 1)


# speedup vs baseline: 2.0476x; 1.5549x over previous
import functools

import jax
import jax.numpy as jnp
from jax import lax
from jax.experimental import pallas as pl
from jax.experimental.pallas import tpu as pltpu
from jax.experimental.pallas import tpu_sc as plsc

_NC = 2
_NS = 16
_NW = _NC * _NS

_VOC = 1000000
_D = 32
_B = 16384
_F = 26

_TOT = _B * _F
_IDX_MINOR = 128
_IDX_ROWS = _TOT // _IDX_MINOR
_ROWS_PER_W = _IDX_ROWS // _NW
_CHUNK_IR = 8
_CHUNK = _CHUNK_IR * _IDX_MINOR
_STEPS = _ROWS_PER_W // _CHUNK_IR
_T1_C = 2048
_T1_GRID = 123
_ST = _T1_C * _T1_GRID
_T1_MAXB = _VOC // _T1_C


def _tc_emb_pack(embT):
    def body(i0, i1, i2, i3, out_ref):
        big = jnp.concatenate(
            [i0[...], i1[...], i2[...], i3[...]], axis=0)
        out_ref[...] = big.T

    def imap(k):
        return lambda c: (0, jnp.minimum(k * _T1_GRID + c, _T1_MAXB))

    return pl.pallas_call(
        body,
        grid=(_T1_GRID,),
        in_specs=[pl.BlockSpec((_D, _T1_C), imap(k)) for k in range(4)],
        out_specs=pl.BlockSpec((_T1_C, 128), lambda c: (c, 0)),
        out_shape=jax.ShapeDtypeStruct((_ST, 128), jnp.float32),
    )(embT, embT, embT, embT)


def _sc_gather(emb, idx2d):
    mesh = plsc.VectorSubcoreMesh(
        core_axis_name="c", subcore_axis_name="s",
        num_cores=_NC, num_subcores=_NS)

    @functools.partial(
        pl.kernel,
        out_type=jax.ShapeDtypeStruct((_TOT, _D), jnp.float32),
        mesh=mesh,
        scratch_types=[
            pltpu.VMEM((2, _CHUNK_IR, _IDX_MINOR), jnp.int32),
            pltpu.VMEM((2, _CHUNK, _D), jnp.float32),
            pltpu.SemaphoreType.DMA,
            pltpu.SemaphoreType.DMA,
            pltpu.SemaphoreType.DMA,
            pltpu.SemaphoreType.DMA,
        ],
        compiler_params=pltpu.CompilerParams(use_tc_tiling_on_sc=False),
    )
    def k(emb_hbm, idx_hbm, out_hbm, idx_v, rows_v, g0, g1, w0, w1):
        wid = lax.axis_index("s") * _NC + lax.axis_index("c")
        base_ir = wid * _ROWS_PER_W
        gsem = (g0, g1)
        wsem = (w0, w1)

        gathers = [None, None]
        writes = [None, None]
        for t in range(_STEPS):
            buf = t % 2
            if writes[buf] is not None:
                writes[buf].wait()
            ir0 = base_ir + t * _CHUNK_IR
            pltpu.sync_copy(idx_hbm.at[pl.ds(ir0, _CHUNK_IR)],
                            idx_v.at[buf])
            gathers[buf] = [
                pltpu.async_copy(
                    emb_hbm.at[idx_v.at[buf, j]],
                    rows_v.at[buf, pl.ds(j * _IDX_MINOR, _IDX_MINOR)],
                    gsem[buf])
                for j in range(_CHUNK_IR)
            ]
            if t >= 1:
                pb = 1 - buf
                for c in gathers[pb]:
                    c.wait()
                pr0 = (base_ir + (t - 1) * _CHUNK_IR) * _IDX_MINOR
                writes[pb] = pltpu.async_copy(
                    rows_v.at[pb], out_hbm.at[pl.ds(pr0, _CHUNK)], wsem[pb])
        last = (_STEPS - 1) % 2
        for c in gathers[last]:
            c.wait()
        lr0 = (base_ir + (_STEPS - 1) * _CHUNK_IR) * _IDX_MINOR
        writes[last] = pltpu.async_copy(
            rows_v.at[last], out_hbm.at[pl.ds(lr0, _CHUNK)], wsem[last])
        writes[0].wait()
        writes[1].wait()

    return k(emb, idx2d)


_TC_BLK_B = 16384
_TC_M = _TC_BLK_B // 4
_TC_NBLK = _B // _TC_BLK_B


def _tc_sigmoid_transpose(flat):
    def body(in_ref, out_ref):
        v2 = in_ref[...].reshape(_TC_M, 128)
        w = v2.T
        parts = [w[32 * k:32 * (k + 1), :] for k in range(4)]
        out_ref[0] = jax.nn.sigmoid(jnp.concatenate(parts, axis=1))

    return pl.pallas_call(
        body,
        grid=(_F, _TC_NBLK),
        in_specs=[pl.BlockSpec((_TC_BLK_B * _D,),
                               lambda f, c: (f * _TC_NBLK + c,))],
        out_specs=pl.BlockSpec((1, _D, _TC_BLK_B), lambda f, c: (f, 0, c)),
        out_shape=jax.ShapeDtypeStruct((_F, _D, _B), jnp.float32),
    )(flat)


@jax.jit
def kernel(x, emb):
    xt = jnp.transpose(x).astype(jnp.int32)
    k = xt // _ST
    xt = (xt - k * _ST) * 4 + k
    idx2d = (xt.reshape(_F, _TC_NBLK, 4, _TC_M)
             .transpose(0, 1, 3, 2)
             .reshape(_IDX_ROWS, _IDX_MINOR))
    emb_pack = _tc_emb_pack(jnp.transpose(emb))
    rows = _sc_gather(emb_pack.reshape(_ST * 4, _D), idx2d)
    out3 = _tc_sigmoid_transpose(rows.reshape(-1))
    return out3.transpose(2, 0, 1)


# speedup vs baseline: 2.3300x; 1.1379x over previous
import functools

import jax
import jax.numpy as jnp
from jax import lax
from jax.experimental import pallas as pl
from jax.experimental.pallas import tpu as pltpu
from jax.experimental.pallas import tpu_sc as plsc

_NC = 2
_NS = 16
_NW = _NC * _NS

_VOC = 1000000
_D = 32
_B = 16384
_F = 26

_TOT = _B * _F
_IDX_MINOR = 128
_IDX_ROWS = _TOT // _IDX_MINOR
_ROWS_PER_W = _IDX_ROWS // _NW
_CHUNK_IR = 8
_CHUNK = _CHUNK_IR * _IDX_MINOR
_STEPS = _ROWS_PER_W // _CHUNK_IR
_T1_C = 4096
_T1_GRID = 62
_ST = _T1_C * _T1_GRID
_T1_MAXB = _VOC // _T1_C


def _tc_emb_pack(embT):
    def body(i0, i1, i2, i3, out_ref):
        big = jnp.concatenate(
            [i0[...], i1[...], i2[...], i3[...]], axis=0)
        out_ref[...] = big.T

    def imap(k):
        return lambda c: (0, jnp.minimum(k * _T1_GRID + c, _T1_MAXB))

    return pl.pallas_call(
        body,
        grid=(_T1_GRID,),
        in_specs=[pl.BlockSpec((_D, _T1_C), imap(k)) for k in range(4)],
        out_specs=pl.BlockSpec((_T1_C, 128), lambda c: (c, 0)),
        out_shape=jax.ShapeDtypeStruct((_ST, 128), jnp.float32),
    )(embT, embT, embT, embT)


def _sc_gather(emb, idx2d):
    mesh = plsc.VectorSubcoreMesh(
        core_axis_name="c", subcore_axis_name="s",
        num_cores=_NC, num_subcores=_NS)

    @functools.partial(
        pl.kernel,
        out_type=jax.ShapeDtypeStruct((_TOT, _D), jnp.float32),
        mesh=mesh,
        scratch_types=[
            pltpu.VMEM((2, _CHUNK_IR, _IDX_MINOR), jnp.int32),
            pltpu.VMEM((2, _CHUNK, _D), jnp.float32),
            pltpu.SemaphoreType.DMA,
            pltpu.SemaphoreType.DMA,
            pltpu.SemaphoreType.DMA,
            pltpu.SemaphoreType.DMA,
        ],
        compiler_params=pltpu.CompilerParams(use_tc_tiling_on_sc=False),
    )
    def k(emb_hbm, idx_hbm, out_hbm, idx_v, rows_v, g0, g1, w0, w1):
        wid = lax.axis_index("s") * _NC + lax.axis_index("c")
        base_ir = wid * _ROWS_PER_W
        gsem = (g0, g1)
        wsem = (w0, w1)

        gathers = [None, None]
        writes = [None, None]
        for t in range(_STEPS):
            buf = t % 2
            if writes[buf] is not None:
                writes[buf].wait()
            ir0 = base_ir + t * _CHUNK_IR
            pltpu.sync_copy(idx_hbm.at[pl.ds(ir0, _CHUNK_IR)],
                            idx_v.at[buf])
            gathers[buf] = [
                pltpu.async_copy(
                    emb_hbm.at[idx_v.at[buf, j]],
                    rows_v.at[buf, pl.ds(j * _IDX_MINOR, _IDX_MINOR)],
                    gsem[buf])
                for j in range(_CHUNK_IR)
            ]
            if t >= 1:
                pb = 1 - buf
                for c in gathers[pb]:
                    c.wait()
                pr0 = (base_ir + (t - 1) * _CHUNK_IR) * _IDX_MINOR
                writes[pb] = pltpu.async_copy(
                    rows_v.at[pb], out_hbm.at[pl.ds(pr0, _CHUNK)], wsem[pb])
        last = (_STEPS - 1) % 2
        for c in gathers[last]:
            c.wait()
        lr0 = (base_ir + (_STEPS - 1) * _CHUNK_IR) * _IDX_MINOR
        writes[last] = pltpu.async_copy(
            rows_v.at[last], out_hbm.at[pl.ds(lr0, _CHUNK)], wsem[last])
        writes[0].wait()
        writes[1].wait()

    return k(emb, idx2d)


_TC_BLK_B = 16384
_TC_M = _TC_BLK_B // 4
_TC_NBLK = _B // _TC_BLK_B


def _tc_sigmoid_transpose(flat):
    def body(in_ref, out_ref):
        v2 = in_ref[...].reshape(_TC_M, 128)
        w = v2.T
        parts = [w[32 * k:32 * (k + 1), :] for k in range(4)]
        out_ref[0] = jax.nn.sigmoid(jnp.concatenate(parts, axis=1))

    return pl.pallas_call(
        body,
        grid=(_F, _TC_NBLK),
        in_specs=[pl.BlockSpec((_TC_BLK_B * _D,),
                               lambda f, c: (f * _TC_NBLK + c,))],
        out_specs=pl.BlockSpec((1, _D, _TC_BLK_B), lambda f, c: (f, 0, c)),
        out_shape=jax.ShapeDtypeStruct((_F, _D, _B), jnp.float32),
    )(flat)


@jax.jit
def kernel(x, emb):
    xt = jnp.transpose(x).astype(jnp.int32)
    k = xt // _ST
    xt = (xt - k * _ST) * 4 + k
    idx2d = (xt.reshape(_F, _TC_NBLK, 4, _TC_M)
             .transpose(0, 1, 3, 2)
             .reshape(_IDX_ROWS, _IDX_MINOR))
    emb_pack = _tc_emb_pack(jnp.transpose(emb))
    rows = _sc_gather(emb_pack.reshape(_ST * 4, _D), idx2d)
    out3 = _tc_sigmoid_transpose(rows.reshape(-1))
    return out3.transpose(2, 0, 1)
